# baseline (speedup 1.0000x reference)
import functools

import jax
import jax.numpy as jnp
from jax import lax
from jax.experimental import pallas as pl
from jax.experimental.pallas import tpu as pltpu
from jax.experimental.pallas import tpu_sc as plsc


_CHUNK = 64


def _make_sc_copy(seq_len, dim):
    info = plsc.get_sparse_core_info()
    nc, ns = info.num_cores, info.num_subcores
    nw = nc * ns
    rows_per_w = seq_len // nw
    nchunks = rows_per_w // _CHUNK
    mesh = plsc.VectorSubcoreMesh(core_axis_name="c", subcore_axis_name="s")

    @functools.partial(
        pl.kernel, mesh=mesh,
        out_type=jax.ShapeDtypeStruct((seq_len, dim), jnp.float32),
        scratch_types=[
            pltpu.VMEM((_CHUNK, dim), jnp.float32),
            pltpu.VMEM((_CHUNK, dim), jnp.float32),
            pltpu.SemaphoreType.DMA,
            pltpu.SemaphoreType.DMA,
        ],
    )
    def sc_copy(w_hbm, out_hbm, buf0, buf1, sem0, sem1):
        wid = lax.axis_index("s") * nc + lax.axis_index("c")
        base = wid * rows_per_w
        bufs = (buf0, buf1)
        sems = (sem0, sem1)
        out_cps = [None] * nchunks
        for c in range(nchunks):
            buf = bufs[c % 2]
            sem = sems[c % 2]
            if c >= 2:
                out_cps[c - 2].wait()
            start = base + c * _CHUNK
            pltpu.async_copy(w_hbm.at[pl.ds(start, _CHUNK)], buf, sem).wait()
            out_cps[c] = pltpu.async_copy(
                buf, out_hbm.at[pl.ds(start, _CHUNK)], sem)
        out_cps[nchunks - 2].wait()
        out_cps[nchunks - 1].wait()

    return sc_copy


def kernel(input_ids, weights):
    seq_len = input_ids.shape[-1]
    dim = weights.shape[-1]
    return _make_sc_copy(seq_len, dim)(weights)
